# baseline (speedup 1.0000x reference)
import functools
import jax
import jax.numpy as jnp
from jax import lax
from jax.experimental import pallas as pl
from jax.experimental.pallas import tpu as pltpu
from jax.experimental.pallas import tpu_sc as plsc

NUM_CLASSES = 20
L = 16
KL = NUM_CLASSES * L
NW = 32
CP = 256


def _inv_norm(ss):
    x = jnp.maximum(ss, jnp.full((L,), 1e-24, jnp.float32))
    i = lax.bitcast_convert_type(x, jnp.int32)
    i = jnp.full((L,), 0x5F3759DF, jnp.int32) - lax.shift_right_logical(
        i, jnp.full((L,), 1, jnp.int32))
    y = lax.bitcast_convert_type(i, jnp.float32)
    half = jnp.full((L,), 0.5, jnp.float32) * x
    threehalf = jnp.full((L,), 1.5, jnp.float32)
    for _ in range(3):
        y = y * (threehalf - half * y * y)
    return y


def _sc_partials(feats, lab):
    bs, c, hw = feats.shape
    pix_per_w = bs * hw // NW
    nchunk = pix_per_w // CP
    wpb = hw // pix_per_w

    mesh = plsc.VectorSubcoreMesh(core_axis_name="c", subcore_axis_name="s")

    @functools.partial(
        pl.kernel, mesh=mesh,
        compiler_params=pltpu.CompilerParams(needs_layout_passes=False),
        out_type=jax.ShapeDtypeStruct((NW, c * KL), jnp.float32),
        scratch_types=[
            pltpu.VMEM((2, c, CP), jnp.float32),
            pltpu.VMEM((2, CP), jnp.int32),
            pltpu.VMEM((c * KL,), jnp.float32),
            pltpu.SemaphoreType.DMA((2,)),
            pltpu.SemaphoreType.DMA((2,)),
        ],
    )
    def k(f_hbm, l_hbm, out_hbm, fbuf, lbuf, acc, fsem, lsem):
        wid = lax.axis_index("s") * 2 + lax.axis_index("c")
        b = wid // wpb
        base = (wid % wpb) * pix_per_w

        zero16 = jnp.zeros((L,), jnp.float32)

        def clr(i, _):
            acc[pl.ds(i * L, L)] = zero16
            return 0
        lax.fori_loop(0, c * NUM_CLASSES, clr, 0)

        lane = lax.iota(jnp.int32, L)

        def start(ch, slot):
            off = base + ch * CP
            pltpu.make_async_copy(f_hbm.at[b, :, pl.ds(off, CP)],
                                  fbuf.at[slot], fsem.at[slot]).start()
            pltpu.make_async_copy(l_hbm.at[b, pl.ds(off, CP)],
                                  lbuf.at[slot], lsem.at[slot]).start()

        def wait(ch, slot):
            off = base + ch * CP
            pltpu.make_async_copy(f_hbm.at[b, :, pl.ds(off, CP)],
                                  fbuf.at[slot], fsem.at[slot]).wait()
            pltpu.make_async_copy(l_hbm.at[b, pl.ds(off, CP)],
                                  lbuf.at[slot], lsem.at[slot]).wait()

        start(0, 0)

        def chunk_body(ch, _):
            slot = lax.rem(ch, 2)

            @pl.when(ch + 1 < nchunk)
            def _():
                start(ch + 1, lax.rem(ch + 1, 2))

            wait(ch, slot)

            def group_body(g, _):
                lab16 = lbuf[slot, pl.ds(g * L, L)]

                parts = [zero16, zero16, zero16, zero16]
                for cc in range(c):
                    v = fbuf[slot, cc, pl.ds(g * L, L)]
                    parts[cc % 4] = parts[cc % 4] + v * v
                ss = (parts[0] + parts[1]) + (parts[2] + parts[3])
                inv = _inv_norm(ss)

                idx0 = lab16 * L + lane

                for cc in range(c):
                    v = fbuf[slot, cc, pl.ds(g * L, L)] * inv
                    plsc.addupdate_scatter(
                        acc.at[pl.ds(cc * KL, KL)], [idx0], v)
                return 0

            lax.fori_loop(0, CP // L, group_body, 0)
            return 0

        lax.fori_loop(0, nchunk, chunk_body, 0)

        pltpu.sync_copy(acc, out_hbm.at[wid])

    return k(feats, lab)


def _tc_finish(partials):
    def body(p_ref, o_ref):
        p = jnp.sum(p_ref[...], axis=0)
        r = lax.broadcasted_iota(jnp.int32, (KL, NUM_CLASSES), 0)
        kk = lax.broadcasted_iota(jnp.int32, (KL, NUM_CLASSES), 1)
        m = jnp.where(lax.shift_right_logical(r, 4) == kk, 1.0, 0.0)
        proto = jax.lax.dot_general(
            p, m, (((1,), (0,)), ((), ())),
            preferred_element_type=jnp.float32)
        pn = jnp.sqrt(jnp.sum(proto * proto, axis=0, keepdims=True))
        o_ref[...] = proto / jnp.maximum(pn, 1e-12)

    nw = partials.shape[0]
    c = partials.shape[1] // KL
    return pl.pallas_call(
        body,
        out_shape=jax.ShapeDtypeStruct((c, NUM_CLASSES), jnp.float32),
    )(partials.reshape(nw, c, KL))


def kernel(features, labels):
    bs, c, h, w = features.shape
    hw = h * w
    feats = features.reshape(bs, c, hw)
    lab = labels.astype(jnp.int32).reshape(bs, hw)

    partials = _sc_partials(feats, lab)
    proto = _tc_finish(partials)
    return proto[:, :NUM_CLASSES - 1]


# speedup vs baseline: 1.7359x; 1.7359x over previous
import functools
import jax
import jax.numpy as jnp
from jax import lax
from jax.experimental import pallas as pl
from jax.experimental.pallas import tpu as pltpu
from jax.experimental.pallas import tpu_sc as plsc

NUM_CLASSES = 20
L = 16
KL = NUM_CLASSES * L
NW = 32
CP = 256


def _inv_norm(ss):
    x = jnp.maximum(ss, jnp.full((L,), 1e-24, jnp.float32))
    i = lax.bitcast_convert_type(x, jnp.int32)
    i = jnp.full((L,), 0x5F3759DF, jnp.int32) - lax.shift_right_logical(
        i, jnp.full((L,), 1, jnp.int32))
    y = lax.bitcast_convert_type(i, jnp.float32)
    half = jnp.full((L,), 0.5, jnp.float32) * x
    threehalf = jnp.full((L,), 1.5, jnp.float32)
    for _ in range(3):
        y = y * (threehalf - half * y * y)
    return y


def _sc_partials(feats, lab):
    bs, c, hw = feats.shape
    pix_per_w = bs * hw // NW
    nchunk = pix_per_w // CP
    wpb = hw // pix_per_w

    mesh = plsc.VectorSubcoreMesh(core_axis_name="c", subcore_axis_name="s")

    @functools.partial(
        pl.kernel, mesh=mesh,
        compiler_params=pltpu.CompilerParams(needs_layout_passes=False),
        out_type=jax.ShapeDtypeStruct((NW, c * KL), jnp.float32),
        scratch_types=[
            pltpu.VMEM((2, c, CP), jnp.float32),
            pltpu.VMEM((2, CP), jnp.int32),
            pltpu.VMEM((c * KL,), jnp.float32),
            pltpu.SemaphoreType.DMA((2,)),
            pltpu.SemaphoreType.DMA((2,)),
        ],
    )
    def k(f_hbm, l_hbm, out_hbm, fbuf, lbuf, acc, fsem, lsem):
        wid = lax.axis_index("s") * 2 + lax.axis_index("c")
        b = wid // wpb
        base = (wid % wpb) * pix_per_w

        zero16 = jnp.zeros((L,), jnp.float32)

        def clr(i, _):
            acc[pl.ds(i * L, L)] = zero16
            return 0
        lax.fori_loop(0, c * NUM_CLASSES, clr, 0)

        lane = lax.iota(jnp.int32, L)

        def start(ch, slot):
            off = base + ch * CP
            pltpu.make_async_copy(f_hbm.at[b, :, pl.ds(off, CP)],
                                  fbuf.at[slot], fsem.at[slot]).start()
            pltpu.make_async_copy(l_hbm.at[b, pl.ds(off, CP)],
                                  lbuf.at[slot], lsem.at[slot]).start()

        def wait(ch, slot):
            off = base + ch * CP
            pltpu.make_async_copy(f_hbm.at[b, :, pl.ds(off, CP)],
                                  fbuf.at[slot], fsem.at[slot]).wait()
            pltpu.make_async_copy(l_hbm.at[b, pl.ds(off, CP)],
                                  lbuf.at[slot], lsem.at[slot]).wait()

        start(0, 0)

        def chunk_body(ch, _):
            slot = lax.rem(ch, 2)

            @pl.when(ch + 1 < nchunk)
            def _():
                start(ch + 1, lax.rem(ch + 1, 2))

            wait(ch, slot)

            def group_body(g, _):
                lab16 = lbuf[slot, pl.ds(g * L, L)]

                parts = [zero16, zero16, zero16, zero16]
                for cc in range(c):
                    v = fbuf[slot, cc, pl.ds(g * L, L)]
                    parts[cc % 4] = parts[cc % 4] + v * v
                ss = (parts[0] + parts[1]) + (parts[2] + parts[3])
                inv = _inv_norm(ss)

                idx0 = lab16 * L + lane

                for cb in range(0, c, 8):
                    vs = [fbuf[slot, cb + j, pl.ds(g * L, L)] * inv
                          for j in range(8)]
                    for j in range(8):
                        plsc.addupdate_scatter(
                            acc.at[pl.ds((cb + j) * KL, KL)], [idx0], vs[j])
                return 0

            lax.fori_loop(0, CP // L, group_body, 0)
            return 0

        lax.fori_loop(0, nchunk, chunk_body, 0)

        pltpu.sync_copy(acc, out_hbm.at[wid])

    return k(feats, lab)


def _tc_finish(partials):
    def body(p_ref, o_ref):
        p = jnp.sum(p_ref[...], axis=0)
        r = lax.broadcasted_iota(jnp.int32, (KL, NUM_CLASSES), 0)
        kk = lax.broadcasted_iota(jnp.int32, (KL, NUM_CLASSES), 1)
        m = jnp.where(lax.shift_right_logical(r, 4) == kk, 1.0, 0.0)
        proto = jax.lax.dot_general(
            p, m, (((1,), (0,)), ((), ())),
            preferred_element_type=jnp.float32)
        pn = jnp.sqrt(jnp.sum(proto * proto, axis=0, keepdims=True))
        o_ref[...] = proto / jnp.maximum(pn, 1e-12)

    nw = partials.shape[0]
    c = partials.shape[1] // KL
    return pl.pallas_call(
        body,
        out_shape=jax.ShapeDtypeStruct((c, NUM_CLASSES), jnp.float32),
    )(partials.reshape(nw, c, KL))


def kernel(features, labels):
    bs, c, h, w = features.shape
    hw = h * w
    feats = features.reshape(bs, c, hw)
    lab = labels.astype(jnp.int32).reshape(bs, hw)

    partials = _sc_partials(feats, lab)
    proto = _tc_finish(partials)
    return proto[:, :NUM_CLASSES - 1]
